# Initial kernel scaffold; baseline (speedup 1.0000x reference)
import functools

import jax
import jax.numpy as jnp
from jax import lax
from jax.experimental import pallas as pl
from jax.experimental.pallas import tpu as pltpu
from jax.experimental.pallas import tpu_sc as plsc

_DIM = 64
_B = 16384 * 50
_NW = 32
_BPW = _B // _NW
_G = 512
_NCHUNK = _BPW // _G

_mesh = plsc.VectorSubcoreMesh(core_axis_name="c", subcore_axis_name="s")


@functools.partial(
    pl.kernel,
    out_type=jax.ShapeDtypeStruct((_B, _DIM), jnp.float32),
    mesh=_mesh,
    scratch_types=[
        pltpu.VMEM((_BPW,), jnp.int32),
        pltpu.VMEM((_G, _DIM), jnp.float32),
        pltpu.SemaphoreType.DMA,
    ],
)
def _embedding_gather(idx_hbm, table_hbm, out_hbm, idx_v, rows_v, gsem):
    wid = lax.axis_index("s") * 2 + lax.axis_index("c")
    base = wid * _BPW
    pltpu.sync_copy(idx_hbm.at[pl.ds(base, _BPW)], idx_v)

    def chunk(c, carry):
        off = pl.multiple_of(c * _G, _G)
        pltpu.async_copy(
            table_hbm.at[idx_v.at[pl.ds(off, _G)]], rows_v, gsem
        ).wait()
        pltpu.sync_copy(rows_v, out_hbm.at[pl.ds(base + off, _G)])
        return carry

    lax.fori_loop(0, _NCHUNK, chunk, 0)


def kernel(token_ids, weight):
    flat = token_ids.reshape(-1).astype(jnp.int32)
    out = _embedding_gather(flat, weight)
    return out.reshape(token_ids.shape + (_DIM,))


# speedup vs baseline: 1.8325x; 1.8325x over previous
import functools

import jax
import jax.numpy as jnp
from jax import lax
from jax.experimental import pallas as pl
from jax.experimental.pallas import tpu as pltpu
from jax.experimental.pallas import tpu_sc as plsc

_DIM = 64
_B = 16384 * 50
_NW = 32
_BPW = _B // _NW
_G = 512
_NCHUNK = _BPW // _G

_mesh = plsc.VectorSubcoreMesh(core_axis_name="c", subcore_axis_name="s")


@functools.partial(
    pl.kernel,
    out_type=jax.ShapeDtypeStruct((_B, _DIM), jnp.float32),
    mesh=_mesh,
    scratch_types=[
        pltpu.VMEM((_BPW,), jnp.int32),
        pltpu.VMEM((_G, _DIM), jnp.float32),
        pltpu.SemaphoreType.DMA,
    ],
    compiler_params=pltpu.CompilerParams(use_tc_tiling_on_sc=False),
)
def _embedding_gather(idx_hbm, table_hbm, out_hbm, idx_v, rows_v, gsem):
    wid = lax.axis_index("s") * 2 + lax.axis_index("c")
    base = wid * _BPW
    pltpu.sync_copy(idx_hbm.at[pl.ds(base, _BPW)], idx_v)

    def chunk(c, carry):
        off = pl.multiple_of(c * _G, _G)
        pltpu.async_copy(
            table_hbm.at[idx_v.at[pl.ds(off, _G)]], rows_v, gsem
        ).wait()
        pltpu.sync_copy(rows_v, out_hbm.at[pl.ds(base + off, _G)])
        return carry

    lax.fori_loop(0, _NCHUNK, chunk, 0)


def kernel(token_ids, weight):
    flat = token_ids.reshape(-1).astype(jnp.int32)
    out = _embedding_gather(flat, weight)
    return out.reshape(token_ids.shape + (_DIM,))


# speedup vs baseline: 1.8726x; 1.0219x over previous
import functools

import jax
import jax.numpy as jnp
from jax import lax
from jax.experimental import pallas as pl
from jax.experimental.pallas import tpu as pltpu
from jax.experimental.pallas import tpu_sc as plsc

_DIM = 64
_B = 16384 * 50
_NW = 32
_BPW = _B // _NW
_G = 256
_NCHUNK = _BPW // _G
_NBUF = 4

_mesh = plsc.VectorSubcoreMesh(core_axis_name="c", subcore_axis_name="s")


@functools.partial(
    pl.kernel,
    out_type=jax.ShapeDtypeStruct((_B, _DIM), jnp.float32),
    mesh=_mesh,
    scratch_types=[
        pltpu.VMEM((_BPW,), jnp.int32),
        *[pltpu.VMEM((_G, _DIM), jnp.float32) for _ in range(_NBUF)],
        *[pltpu.SemaphoreType.DMA for _ in range(_NBUF)],
    ],
    compiler_params=pltpu.CompilerParams(use_tc_tiling_on_sc=False),
)
def _embedding_gather(idx_hbm, table_hbm, out_hbm, idx_v, *bufs_and_sems):
    rows = bufs_and_sems[:_NBUF]
    sems = bufs_and_sems[_NBUF:]
    wid = lax.axis_index("s") * 2 + lax.axis_index("c")
    base = wid * _BPW
    pltpu.sync_copy(idx_hbm.at[pl.ds(base, _BPW)], idx_v)

    def start_gather(c, b):
        off = pl.multiple_of(c * _G, _G)
        pltpu.async_copy(
            table_hbm.at[idx_v.at[pl.ds(off, _G)]], rows[b], sems[b]
        )

    def finish_chunk(c, b):
        pltpu.make_async_copy(
            table_hbm.at[idx_v.at[pl.ds(0, _G)]], rows[b], sems[b]
        ).wait()
        off = pl.multiple_of(c * _G, _G)
        pltpu.sync_copy(rows[b], out_hbm.at[pl.ds(base + off, _G)])

    for b in range(_NBUF):
        start_gather(b, b)

    def group(cg, carry):
        for b in range(_NBUF):
            c = cg * _NBUF + b
            finish_chunk(c, b)
            start_gather(c + _NBUF, b)
        return carry

    lax.fori_loop(0, _NCHUNK // _NBUF - 1, group, 0)
    for b in range(_NBUF):
        finish_chunk(_NCHUNK - _NBUF + b, b)


def kernel(token_ids, weight):
    flat = token_ids.reshape(-1).astype(jnp.int32)
    out = _embedding_gather(flat, weight)
    return out.reshape(token_ids.shape + (_DIM,))
